# baseline (speedup 1.0000x reference)
import jax
import jax.numpy as jnp
from jax.experimental import pallas as pl
from jax.experimental.pallas import tpu as pltpu

_KC = 2048


def _vq_kernel(x_ref, cb_ref, out_ref):
    x = x_ref[...]
    xm2 = x * (-2.0)
    in_sq = jnp.sum(x * x, axis=1)[None, :]
    K = cb_ref.shape[0]

    run_min = None
    run_idx = None
    for c in range(K // _KC):
        cbc = cb_ref[c * _KC:(c + 1) * _KC, :]
        cbsq = jnp.sum(cbc * cbc, axis=1, keepdims=True)
        mmT = jax.lax.dot_general(
            cbc, xm2,
            dimension_numbers=(((1,), (1,)), ((), ())),
            preferred_element_type=jnp.float32,
        )
        dist = (cbsq + in_sq) + mmT
        SL = 64
        run_v = dist[0:SL, :]
        run_j = jnp.zeros(run_v.shape, jnp.int32)
        for j in range(1, _KC // SL):
            dj = dist[j * SL:(j + 1) * SL, :]
            upd = dj < run_v
            run_v = jnp.where(upd, dj, run_v)
            run_j = jnp.where(upd, j, run_j)
        kk = (run_j * SL + (c * _KC)) + jax.lax.broadcasted_iota(
            jnp.int32, run_v.shape, 0)
        v = run_v
        while v.shape[0] > 1:
            h = v.shape[0] // 2
            a, b = v[:h, :], v[h:, :]
            ka, kb = kk[:h, :], kk[h:, :]
            m = (a < b) | ((a == b) & (ka < kb))
            v = jnp.where(m, a, b)
            kk = jnp.where(m, ka, kb)
        loc_min, loc_idx = v, kk
        if run_min is None:
            run_min, run_idx = loc_min, loc_idx
        else:
            upd = loc_min < run_min
            run_min = jnp.where(upd, loc_min, run_min)
            run_idx = jnp.where(upd, loc_idx, run_idx)
    Nv = out_ref.shape[1]
    for b in range(out_ref.shape[0]):
        out_ref[b:b + 1, :] = run_idx[:, b * Nv:(b + 1) * Nv]


def kernel(z_e_x, codebook):
    Bv, Nv, D = z_e_x.shape
    K = codebook.shape[0]
    M = Bv * Nv
    flat = z_e_x.reshape(M, D).astype(jnp.float32)
    cb = codebook.astype(jnp.float32)

    idx = pl.pallas_call(
        _vq_kernel,
        grid=(1,),
        in_specs=[
            pl.BlockSpec((M, D), lambda i: (0, 0)),
            pl.BlockSpec((K, D), lambda i: (0, 0)),
        ],
        out_specs=pl.BlockSpec((Bv, Nv), lambda i: (0, 0)),
        out_shape=jax.ShapeDtypeStruct((Bv, Nv), jnp.int32),
    )(flat, cb)
    return idx


# speedup vs baseline: 1.1079x; 1.1079x over previous
import jax
import jax.numpy as jnp
from jax.experimental import pallas as pl
from jax.experimental.pallas import tpu as pltpu

_KC = 512


def _vq_kernel(x_ref, cb_ref, out_ref):
    x = x_ref[...]
    xm2 = x * (-2.0)
    in_sq = jnp.sum(x * x, axis=1)[None, :]
    K = cb_ref.shape[0]

    run_min = None
    run_idx = None
    for c in range(K // _KC):
        cbc = cb_ref[c * _KC:(c + 1) * _KC, :]
        cbsq = jnp.sum(cbc * cbc, axis=1, keepdims=True)
        mmT = jax.lax.dot_general(
            cbc, xm2,
            dimension_numbers=(((1,), (1,)), ((), ())),
            preferred_element_type=jnp.float32,
        )
        dist = (cbsq + in_sq) + mmT
        loc_min = jnp.min(dist, axis=0)[None, :]
        loc_idx = jnp.argmin(dist, axis=0).astype(jnp.int32)[None, :] + (c * _KC)
        if run_min is None:
            run_min, run_idx = loc_min, loc_idx
        else:
            upd = loc_min < run_min
            run_min = jnp.where(upd, loc_min, run_min)
            run_idx = jnp.where(upd, loc_idx, run_idx)
    Nv = out_ref.shape[1]
    for b in range(out_ref.shape[0]):
        out_ref[b:b + 1, :] = run_idx[:, b * Nv:(b + 1) * Nv]


def kernel(z_e_x, codebook):
    Bv, Nv, D = z_e_x.shape
    K = codebook.shape[0]
    M = Bv * Nv
    flat = z_e_x.reshape(M, D).astype(jnp.float32)
    cb = codebook.astype(jnp.float32)

    idx = pl.pallas_call(
        _vq_kernel,
        grid=(1,),
        in_specs=[
            pl.BlockSpec((M, D), lambda i: (0, 0)),
            pl.BlockSpec((K, D), lambda i: (0, 0)),
        ],
        out_specs=pl.BlockSpec((Bv, Nv), lambda i: (0, 0)),
        out_shape=jax.ShapeDtypeStruct((Bv, Nv), jnp.int32),
    )(flat, cb)
    return idx
